# baseline (speedup 1.0000x reference)
import dataclasses

import jax
import jax.numpy as jnp
from jax import lax
from jax.experimental import pallas as pl
from jax.experimental.pallas import tpu as pltpu
from jax.experimental.pallas import tpu_sc as plsc

BATCH = 16384
F = 16
LANES = 128
NC = 2
NS = 16
NW = NC * NS
RPW = BATCH // NW
GRP = 8
NGROUPS = RPW // GRP
NBUF = 3
TRIPLES = (NGROUPS - 1) // NBUF

_DNUMS = lax.GatherDimensionNumbers(
    offset_dims=(), collapsed_slice_dims=(0,), start_index_map=(0,))


def _bcast_lane(v, e):
    idx = jnp.full((F, 1), e, jnp.int32)
    return lax.gather(v, idx, dimension_numbers=_DNUMS, slice_sizes=(1,),
                      mode=lax.GatherScatterMode.PROMISE_IN_BOUNDS)


def _gmf_sc(user2d, item2d, ue_t, ie_t, params, dummy):
    mesh = plsc.VectorSubcoreMesh(core_axis_name="c", subcore_axis_name="s")
    cp = pltpu.CompilerParams()
    if "needs_layout_passes" in pltpu.CompilerParams.__dataclass_fields__:
        cp = dataclasses.replace(cp, needs_layout_passes=False)

    @pl.kernel(
        compiler_params=cp,
        out_type=jax.ShapeDtypeStruct((BATCH,), jnp.float32),
        mesh=mesh,
        scratch_types=[
            pltpu.VMEM((RPW + F,), jnp.int32),
            pltpu.VMEM((RPW + F,), jnp.int32),
            pltpu.VMEM((F,), jnp.int32),
            pltpu.VMEM((NBUF, GRP, F, LANES), jnp.float32),
            pltpu.VMEM((NBUF, GRP, F, LANES), jnp.float32),
            pltpu.VMEM((F,), jnp.float32),
            pltpu.VMEM((RPW,), jnp.float32),
            pltpu.VMEM((2, F), jnp.float32),
            pltpu.SemaphoreType.DMA,
            pltpu.SemaphoreType.DMA,
            pltpu.SemaphoreType.DMA,
        ],
    )
    def k(user_hbm, item_hbm, ue_hbm, ie_hbm, par_hbm, dummy_hbm, out_hbm,
          u_idx, i_idx, fidx_v, ublks, iblks, accv, out_v, par_v,
          sem0, sem1, sem2):
        wid = lax.axis_index("s") * NC + lax.axis_index("c")
        pltpu.sync_copy(user_hbm.at[wid], u_idx.at[pl.ds(0, RPW)])
        pltpu.sync_copy(item_hbm.at[wid], i_idx.at[pl.ds(0, RPW)])
        pltpu.sync_copy(par_hbm, par_v)
        lanes = lax.iota(jnp.int32, F)
        fidx_v[...] = lanes

        sems = (sem0, sem1, sem2)

        def fire(g, buf):
            base = pl.multiple_of(g * GRP, GRP)
            ub = u_idx[pl.ds(base, F)] & ~(LANES - 1)
            ib = i_idx[pl.ds(base, F)] & ~(LANES - 1)
            for e in range(GRP):
                mask = lanes == e
                bu = pl.multiple_of(
                    jnp.sum(jnp.where(mask, ub, 0)), LANES)
                bi = pl.multiple_of(
                    jnp.sum(jnp.where(mask, ib, 0)), LANES)
                pltpu.async_copy(ue_hbm.at[fidx_v, pl.ds(bu, LANES)],
                                 ublks.at[buf, e], sems[buf])
                pltpu.async_copy(ie_hbm.at[fidx_v, pl.ds(bi, LANES)],
                                 iblks.at[buf, e], sems[buf])

        def drain(buf):
            pltpu.make_async_copy(dummy_hbm, ublks.at[buf], sems[buf]).wait()
            pltpu.make_async_copy(dummy_hbm, iblks.at[buf], sems[buf]).wait()

        wvec = par_v[0]
        bvec = par_v[1]
        accv[...] = bvec

        def compute(g, buf, pos_dyn):
            base = pl.multiple_of(g * GRP, GRP)
            ulu = u_idx[pl.ds(base, F)] & (LANES - 1)
            uli = i_idx[pl.ds(base, F)] & (LANES - 1)
            a = accv[...]
            for e in range(GRP):
                ucol = plsc.load_gather(
                    ublks.at[buf, e], [lanes, _bcast_lane(ulu, e)])
                icol = plsc.load_gather(
                    iblks.at[buf, e], [lanes, _bcast_lane(uli, e)])
                s = jnp.sum(ucol * icol * wvec)
                if pos_dyn is None:
                    pos = jnp.full((F,), GRP + e, jnp.int32)
                else:
                    pos = jnp.full((F,), pos_dyn + e, jnp.int32)
                a = a + jnp.where(lanes == pos, s, 0.0)
            accv[...] = a

        def flush(g):
            base = pl.multiple_of((g - 1) * GRP, 2 * GRP)
            out_v[pl.ds(base, F)] = accv[...]
            accv[...] = bvec

        fire(0, 0)
        fire(1, 1)
        fire(2, 2)

        @pl.loop(0, TRIPLES)
        def _(qq):
            for j in range(NBUF):
                g = qq * NBUF + j
                drain(j)
                compute(g, j, (g % 2) * GRP)

                @pl.when(g + NBUF < NGROUPS)
                def _():
                    fire(g + NBUF, j)

                @pl.when(g % 2 == 1)
                def _():
                    flush(g)

        drain(0)
        compute(NGROUPS - 1, 0, None)
        base = pl.multiple_of((NGROUPS - 2) * GRP, 2 * GRP)
        out_v[pl.ds(base, F)] = accv[...]

        pltpu.sync_copy(out_v, out_hbm.at[pl.ds(wid * RPW, RPW)])

    return k(user2d, item2d, ue_t, ie_t, params, dummy)


@jax.jit
def kernel(user, item, user_emb, item_emb, W, b):
    user2d = user.astype(jnp.int32).reshape(NW, RPW)
    item2d = item.astype(jnp.int32).reshape(NW, RPW)
    ue_t = user_emb.T
    ie_t = item_emb.T
    params = jnp.concatenate(
        [W.reshape(1, F), jnp.broadcast_to(b.reshape(1, 1), (1, F))], axis=0)
    dummy = jnp.zeros((GRP, F, LANES), jnp.float32)
    return _gmf_sc(user2d, item2d, ue_t, ie_t, params, dummy)
